# baseline (speedup 1.0000x reference)
import functools

import jax
import jax.numpy as jnp
from jax import lax
from jax.experimental import pallas as pl
from jax.experimental.pallas import tpu as pltpu
from jax.experimental.pallas import tpu_sc as plsc

_B, _N, _C = 4, 10000, 128
_E = 320000
_NW = 32
_EPT = _E // _NW
_CH = 80
_NCH = _EPT // _CH
_PAIRS = (_NCH - 1) // 2
_D = _B * _C

_RATIO = 0.5


def _vsqrt(v):
    i = lax.bitcast_convert_type(v, jnp.int32)
    i = jnp.int32(0x5F3759DF) - lax.shift_right_arithmetic(i, 1)
    y = lax.bitcast_convert_type(i, jnp.float32)
    for _ in range(3):
        y = y * (1.5 - 0.5 * v * y * y)
    return v * y


def _sc_importance(xt, row, col):
    mesh = plsc.VectorSubcoreMesh(core_axis_name="c", subcore_axis_name="s")

    @functools.partial(
        pl.kernel,
        out_type=jax.ShapeDtypeStruct((_NW, _N), jnp.float32),
        mesh=mesh,
        scratch_types=[
            pltpu.VMEM((_EPT,), jnp.int32),
            pltpu.VMEM((_EPT,), jnp.int32),
            pltpu.VMEM((2 * _CH, _D), jnp.bfloat16),
            pltpu.VMEM((2 * _CH, _D), jnp.bfloat16),
            pltpu.VMEM((_N,), jnp.float32),
            pltpu.VMEM((4 * _CH, 17), jnp.float32),
            pltpu.SemaphoreType.DMA,
            pltpu.SemaphoreType.DMA,
        ],
        compiler_params=pltpu.CompilerParams(
            use_tc_tiling_on_sc=False, needs_layout_passes=False
        ),
    )
    def k(xt_hbm, row_hbm, col_hbm, out_hbm,
          idxr, idxc, buf0, buf1, acc, t, s0, s1):
        wid = lax.axis_index("s") * 2 + lax.axis_index("c")
        ebase = wid * _EPT
        pltpu.sync_copy(row_hbm.at[pl.ds(ebase, _EPT)], idxr)
        pltpu.sync_copy(col_hbm.at[pl.ds(ebase, _EPT)], idxc)

        zero16 = jnp.zeros((16,), jnp.float32)

        def zbody(i, carry):
            acc[pl.ds(i * 16, 16)] = zero16
            return carry

        lax.fori_loop(0, _N // 16, zbody, 0)

        lanes = lax.iota(jnp.int32, 16)

        def fire(ci, buf, sem):
            pltpu.async_copy(xt_hbm.at[idxr.at[pl.ds(ci * _CH, _CH)]],
                             buf.at[pl.ds(0, _CH)], sem)
            pltpu.async_copy(xt_hbm.at[idxc.at[pl.ds(ci * _CH, _CH)]],
                             buf.at[pl.ds(_CH, _CH)], sem)

        def drain(buf, sem):
            pltpu.make_async_copy(xt_hbm.at[pl.ds(0, 2 * _CH)], buf, sem).wait()

        def compute(buf, ci):
            def ebody(e, carry):
                for b in range(_B):
                    ps = []
                    for g2 in range(_C // 32):
                        off = b * _C + g2 * 32
                        d = buf[e, pl.ds(off, 32)] - buf[_CH + e, pl.ds(off, 32)]
                        d0, d1 = plsc.unpack(d, format=plsc.PackFormat.INTERLEAVED)
                        ps.append(d0 * d0)
                        ps.append(d1 * d1)
                    while len(ps) > 1:
                        ps = [ps[i] + ps[i + 1] for i in range(0, len(ps), 2)]
                    t[b * _CH + e, pl.ds(0, 16)] = ps[0]
                return carry

            lax.fori_loop(0, _CH, ebody, 0, unroll=2)

            for g in range(_CH // 16):
                s = zero16
                for b in range(_B):
                    rows = b * _CH + g * 16 + lanes
                    gs = [
                        plsc.load_gather(t, [rows, jnp.full((16,), l, jnp.int32)])
                        for l in range(16)
                    ]
                    while len(gs) > 1:
                        gs = [gs[i] + gs[i + 1] for i in range(0, len(gs), 2)]
                    s = s + _vsqrt(gs[0] + 1e-6)
                nid = idxr[pl.ds(ci * _CH + g * 16, 16)]
                plsc.addupdate_scatter(acc, [nid], s * 0.25)

        fire(0, buf0, s0)

        def pair(j, carry):
            ci0 = j * 2
            drain(buf0, s0)
            fire(ci0 + 1, buf1, s1)
            compute(buf0, ci0)
            drain(buf1, s1)
            fire(ci0 + 2, buf0, s0)
            compute(buf1, ci0 + 1)
            return carry

        lax.fori_loop(0, _PAIRS, pair, 0)
        drain(buf0, s0)
        compute(buf0, _NCH - 1)

        pltpu.sync_copy(acc, out_hbm.at[wid])

    return k(xt, row, col)


def _tc_pool(p_even, p_odd, x4):
    kb = 1000
    grid = (_N // 2 // kb,)

    def body(pe_ref, po_ref, x_ref, o_ref):
        we = 1.0 / (1.0 + jnp.exp(-jnp.sum(pe_ref[...], axis=1)))
        wo = 1.0 / (1.0 + jnp.exp(-jnp.sum(po_ref[...], axis=1)))
        xb = x_ref[...]
        o_ref[...] = (
            xb[:, :, 0, :] * we[None, :, None] + xb[:, :, 1, :] * wo[None, :, None]
        ) * 0.5

    return pl.pallas_call(
        body,
        grid=grid,
        in_specs=[
            pl.BlockSpec((kb, _NW), lambda i: (i, 0)),
            pl.BlockSpec((kb, _NW), lambda i: (i, 0)),
            pl.BlockSpec((_B, kb, 2, _C), lambda i: (0, i, 0, 0)),
        ],
        out_specs=pl.BlockSpec((_B, kb, _C), lambda i: (0, i, 0)),
        out_shape=jax.ShapeDtypeStruct((_B, _N // 2, _C), jnp.float32),
    )(p_even, p_odd, x4)


def kernel(x, edge_index):
    B, N, C = x.shape
    num_keep = max(1, int(N * _RATIO))
    row = edge_index[0].astype(jnp.int32)
    col = edge_index[1].astype(jnp.int32)

    xt = x.transpose(1, 0, 2).reshape(N, B * C).astype(jnp.bfloat16)
    partials = _sc_importance(xt, row, col)

    x4 = x.reshape(B, N // 2, 2, C)
    p_even = partials[:, 0::2].T
    p_odd = partials[:, 1::2].T
    x_pooled = _tc_pool(p_even, p_odd, x4)

    idx = jnp.arange(num_keep, dtype=jnp.int64)
    left = idx[:-1]
    right = idx[1:]
    new_edge_index = jnp.concatenate(
        [jnp.stack([left, right], axis=0), jnp.stack([right, left], axis=0)], axis=1
    )
    return (x_pooled, new_edge_index)


# speedup vs baseline: 1.1267x; 1.1267x over previous
import functools

import jax
import jax.numpy as jnp
from jax import lax
from jax.experimental import pallas as pl
from jax.experimental.pallas import tpu as pltpu
from jax.experimental.pallas import tpu_sc as plsc

_B, _N, _C = 4, 10000, 128
_E = 320000
_NW = 32
_EPT = _E // _NW
_CH = 16
_NCH = _EPT // _CH
_PAIRS = (_NCH - 1) // 2
_D = _B * _C

_RATIO = 0.5


def _vsqrt(v):
    i = lax.bitcast_convert_type(v, jnp.int32)
    i = jnp.int32(0x5F3759DF) - lax.shift_right_arithmetic(i, 1)
    y = lax.bitcast_convert_type(i, jnp.float32)
    for _ in range(3):
        y = y * (1.5 - 0.5 * v * y * y)
    return v * y


def _sc_importance(xt, row, col):
    mesh = plsc.VectorSubcoreMesh(core_axis_name="c", subcore_axis_name="s")

    @functools.partial(
        pl.kernel,
        out_type=jax.ShapeDtypeStruct((_NW, _N), jnp.float32),
        mesh=mesh,
        scratch_types=[
            pltpu.VMEM((_EPT,), jnp.int32),
            pltpu.VMEM((_EPT,), jnp.int32),
            pltpu.VMEM((2 * _CH, _D), jnp.float8_e4m3fn),
            pltpu.VMEM((2 * _CH, _D), jnp.float8_e4m3fn),
            pltpu.VMEM((_N,), jnp.float32),
            pltpu.VMEM((4 * _CH, 17), jnp.float32),
            pltpu.SemaphoreType.DMA,
            pltpu.SemaphoreType.DMA,
        ],
        compiler_params=pltpu.CompilerParams(
            use_tc_tiling_on_sc=False, needs_layout_passes=False
        ),
    )
    def k(xt_hbm, row_hbm, col_hbm, out_hbm,
          idxr, idxc, buf0, buf1, acc, t, s0, s1):
        wid = lax.axis_index("s") * 2 + lax.axis_index("c")
        ebase = wid * _EPT
        pltpu.sync_copy(row_hbm.at[pl.ds(ebase, _EPT)], idxr)
        pltpu.sync_copy(col_hbm.at[pl.ds(ebase, _EPT)], idxc)

        zero16 = jnp.zeros((16,), jnp.float32)

        def zbody(i, carry):
            acc[pl.ds(i * 16, 16)] = zero16
            return carry

        lax.fori_loop(0, _N // 16, zbody, 0)

        lanes = lax.iota(jnp.int32, 16)

        def fire(ci, buf, sem):
            pltpu.async_copy(xt_hbm.at[idxr.at[pl.ds(ci * _CH, _CH)]],
                             buf.at[pl.ds(0, _CH)], sem)
            pltpu.async_copy(xt_hbm.at[idxc.at[pl.ds(ci * _CH, _CH)]],
                             buf.at[pl.ds(_CH, _CH)], sem)

        def drain(buf, sem):
            pltpu.make_async_copy(xt_hbm.at[pl.ds(0, 2 * _CH)], buf, sem).wait()

        def compute(buf, ci):
            def ebody(e, carry):
                for b in range(_B):
                    ps = []
                    for g2 in range(_C // 64):
                        off = b * _C + g2 * 64
                        vr = buf[e, pl.ds(off, 64)]
                        vc = buf[_CH + e, pl.ds(off, 64)]
                        r0, r1 = plsc.unpack(
                            vr, format=plsc.PackFormat.INTERLEAVED,
                            preferred_element_type=jnp.bfloat16)
                        c0, c1 = plsc.unpack(
                            vc, format=plsc.PackFormat.INTERLEAVED,
                            preferred_element_type=jnp.bfloat16)
                        for dd in (r0 - c0, r1 - c1):
                            d0, d1 = plsc.unpack(
                                dd, format=plsc.PackFormat.INTERLEAVED)
                            ps.append(d0 * d0)
                            ps.append(d1 * d1)
                    while len(ps) > 1:
                        ps = [ps[i] + ps[i + 1] for i in range(0, len(ps), 2)]
                    t[b * _CH + e, pl.ds(0, 16)] = ps[0]
                return carry

            lax.fori_loop(0, _CH, ebody, 0, unroll=2)

            for g in range(_CH // 16):
                s = zero16
                for b in range(_B):
                    rows = b * _CH + g * 16 + lanes
                    gs = [
                        plsc.load_gather(t, [rows, jnp.full((16,), l, jnp.int32)])
                        for l in range(16)
                    ]
                    while len(gs) > 1:
                        gs = [gs[i] + gs[i + 1] for i in range(0, len(gs), 2)]
                    s = s + _vsqrt(gs[0] + 1e-6)
                nid = idxr[pl.ds(ci * _CH + g * 16, 16)]
                plsc.addupdate_scatter(acc, [nid], s * 0.25)

        fire(0, buf0, s0)

        def pair(j, carry):
            ci0 = j * 2
            drain(buf0, s0)
            fire(ci0 + 1, buf1, s1)
            compute(buf0, ci0)
            drain(buf1, s1)
            fire(ci0 + 2, buf0, s0)
            compute(buf1, ci0 + 1)
            return carry

        lax.fori_loop(0, _PAIRS, pair, 0)
        drain(buf0, s0)
        compute(buf0, _NCH - 1)

        pltpu.sync_copy(acc, out_hbm.at[wid])

    return k(xt, row, col)


def _tc_pool(p_even, p_odd, x4):
    kb = 1000
    grid = (_N // 2 // kb,)

    def body(pe_ref, po_ref, x_ref, o_ref):
        we = 1.0 / (1.0 + jnp.exp(-jnp.sum(pe_ref[...], axis=1)))
        wo = 1.0 / (1.0 + jnp.exp(-jnp.sum(po_ref[...], axis=1)))
        xb = x_ref[...]
        o_ref[...] = (
            xb[:, :, 0, :] * we[None, :, None] + xb[:, :, 1, :] * wo[None, :, None]
        ) * 0.5

    return pl.pallas_call(
        body,
        grid=grid,
        in_specs=[
            pl.BlockSpec((kb, _NW), lambda i: (i, 0)),
            pl.BlockSpec((kb, _NW), lambda i: (i, 0)),
            pl.BlockSpec((_B, kb, 2, _C), lambda i: (0, i, 0, 0)),
        ],
        out_specs=pl.BlockSpec((_B, kb, _C), lambda i: (0, i, 0)),
        out_shape=jax.ShapeDtypeStruct((_B, _N // 2, _C), jnp.float32),
    )(p_even, p_odd, x4)


def kernel(x, edge_index):
    B, N, C = x.shape
    num_keep = max(1, int(N * _RATIO))
    row = edge_index[0].astype(jnp.int32)
    col = edge_index[1].astype(jnp.int32)

    xt = x.transpose(1, 0, 2).reshape(N, B * C).astype(jnp.float8_e4m3fn)
    partials = _sc_importance(xt, row, col)

    x4 = x.reshape(B, N // 2, 2, C)
    p_even = partials[:, 0::2].T
    p_odd = partials[:, 1::2].T
    x_pooled = _tc_pool(p_even, p_odd, x4)

    idx = jnp.arange(num_keep, dtype=jnp.int64)
    left = idx[:-1]
    right = idx[1:]
    new_edge_index = jnp.concatenate(
        [jnp.stack([left, right], axis=0), jnp.stack([right, left], axis=0)], axis=1
    )
    return (x_pooled, new_edge_index)


# speedup vs baseline: 1.1342x; 1.0067x over previous
import functools

import jax
import jax.numpy as jnp
from jax import lax
from jax.experimental import pallas as pl
from jax.experimental.pallas import tpu as pltpu
from jax.experimental.pallas import tpu_sc as plsc

_B, _N, _C = 4, 10000, 128
_E = 320000
_NW = 32
_EPT = _E // _NW
_CH = 16
_NCH = _EPT // _CH
_PAIRS = (_NCH - 1) // 2
_D = _B * _C

_RATIO = 0.5


def _vsqrt(v):
    i = lax.bitcast_convert_type(v, jnp.int32)
    i = jnp.int32(0x5F3759DF) - lax.shift_right_arithmetic(i, 1)
    y = lax.bitcast_convert_type(i, jnp.float32)
    for _ in range(2):
        y = y * (1.5 - 0.5 * v * y * y)
    return v * y


def _sc_importance(xt, row, col):
    mesh = plsc.VectorSubcoreMesh(core_axis_name="c", subcore_axis_name="s")

    @functools.partial(
        pl.kernel,
        out_type=jax.ShapeDtypeStruct((_NW, _N), jnp.float32),
        mesh=mesh,
        scratch_types=[
            pltpu.VMEM((_EPT,), jnp.int32),
            pltpu.VMEM((_EPT,), jnp.int32),
            pltpu.VMEM((2 * _CH, _D), jnp.float8_e4m3fn),
            pltpu.VMEM((2 * _CH, _D), jnp.float8_e4m3fn),
            pltpu.VMEM((_N,), jnp.float32),
            pltpu.VMEM((4 * _CH, 17), jnp.float32),
            pltpu.SemaphoreType.DMA,
            pltpu.SemaphoreType.DMA,
        ],
        compiler_params=pltpu.CompilerParams(
            use_tc_tiling_on_sc=False, needs_layout_passes=False
        ),
    )
    def k(xt_hbm, row_hbm, col_hbm, out_hbm,
          idxr, idxc, buf0, buf1, acc, t, s0, s1):
        wid = lax.axis_index("s") * 2 + lax.axis_index("c")
        ebase = wid * _EPT
        pltpu.sync_copy(row_hbm.at[pl.ds(ebase, _EPT)], idxr)
        pltpu.sync_copy(col_hbm.at[pl.ds(ebase, _EPT)], idxc)

        zero16 = jnp.zeros((16,), jnp.float32)

        def zbody(i, carry):
            acc[pl.ds(i * 16, 16)] = zero16
            return carry

        lax.fori_loop(0, _N // 16, zbody, 0)

        lanes = lax.iota(jnp.int32, 16)

        def fire(ci, buf, sem):
            pltpu.async_copy(xt_hbm.at[idxr.at[pl.ds(ci * _CH, _CH)]],
                             buf.at[pl.ds(0, _CH)], sem)
            pltpu.async_copy(xt_hbm.at[idxc.at[pl.ds(ci * _CH, _CH)]],
                             buf.at[pl.ds(_CH, _CH)], sem)

        def drain(buf, sem):
            pltpu.make_async_copy(xt_hbm.at[pl.ds(0, 2 * _CH)], buf, sem).wait()

        def compute(buf, ci):
            def ebody(e, carry):
                for b in range(_B):
                    ps = []
                    for g2 in range(_C // 64):
                        off = b * _C + g2 * 64
                        vr = buf[e, pl.ds(off, 64)]
                        vc = buf[_CH + e, pl.ds(off, 64)]
                        r0, r1 = plsc.unpack(
                            vr, format=plsc.PackFormat.INTERLEAVED,
                            preferred_element_type=jnp.bfloat16)
                        c0, c1 = plsc.unpack(
                            vc, format=plsc.PackFormat.INTERLEAVED,
                            preferred_element_type=jnp.bfloat16)
                        for dd in (r0 - c0, r1 - c1):
                            iv = plsc.bitcast(dd, jnp.int32)
                            d0 = plsc.bitcast(
                                lax.shift_left(iv, jnp.int32(16)), jnp.float32)
                            d1 = plsc.bitcast(iv & jnp.int32(-65536), jnp.float32)
                            ps.append(d0 * d0)
                            ps.append(d1 * d1)
                    while len(ps) > 1:
                        ps = [ps[i] + ps[i + 1] for i in range(0, len(ps), 2)]
                    t[b * _CH + e, pl.ds(0, 16)] = ps[0]
                return carry

            lax.fori_loop(0, _CH, ebody, 0, unroll=2)

            for g in range(_CH // 16):
                s = zero16
                for b in range(_B):
                    rows = b * _CH + g * 16 + lanes
                    gs = [
                        plsc.load_gather(t, [rows, jnp.full((16,), l, jnp.int32)])
                        for l in range(16)
                    ]
                    while len(gs) > 1:
                        gs = [gs[i] + gs[i + 1] for i in range(0, len(gs), 2)]
                    s = s + _vsqrt(gs[0] + 1e-6)
                nid = idxr[pl.ds(ci * _CH + g * 16, 16)]
                plsc.addupdate_scatter(acc, [nid], s * 0.25)

        fire(0, buf0, s0)

        def pair(j, carry):
            ci0 = j * 2
            drain(buf0, s0)
            fire(ci0 + 1, buf1, s1)
            compute(buf0, ci0)
            drain(buf1, s1)
            fire(ci0 + 2, buf0, s0)
            compute(buf1, ci0 + 1)
            return carry

        lax.fori_loop(0, _PAIRS, pair, 0)
        drain(buf0, s0)
        compute(buf0, _NCH - 1)

        pltpu.sync_copy(acc, out_hbm.at[wid])

    return k(xt, row, col)


def _tc_pool(p_even, p_odd, x4):
    kb = 1000
    grid = (_N // 2 // kb,)

    def body(pe_ref, po_ref, x_ref, o_ref):
        we = 1.0 / (1.0 + jnp.exp(-jnp.sum(pe_ref[...], axis=1)))
        wo = 1.0 / (1.0 + jnp.exp(-jnp.sum(po_ref[...], axis=1)))
        xb = x_ref[...]
        o_ref[...] = (
            xb[:, :, 0, :] * we[None, :, None] + xb[:, :, 1, :] * wo[None, :, None]
        ) * 0.5

    return pl.pallas_call(
        body,
        grid=grid,
        in_specs=[
            pl.BlockSpec((kb, _NW), lambda i: (i, 0)),
            pl.BlockSpec((kb, _NW), lambda i: (i, 0)),
            pl.BlockSpec((_B, kb, 2, _C), lambda i: (0, i, 0, 0)),
        ],
        out_specs=pl.BlockSpec((_B, kb, _C), lambda i: (0, i, 0)),
        out_shape=jax.ShapeDtypeStruct((_B, _N // 2, _C), jnp.float32),
    )(p_even, p_odd, x4)


def kernel(x, edge_index):
    B, N, C = x.shape
    num_keep = max(1, int(N * _RATIO))
    row = edge_index[0].astype(jnp.int32)
    col = edge_index[1].astype(jnp.int32)

    xt = x.transpose(1, 0, 2).reshape(N, B * C).astype(jnp.float8_e4m3fn)
    partials = _sc_importance(xt, row, col)

    x4 = x.reshape(B, N // 2, 2, C)
    p_even = partials[:, 0::2].T
    p_odd = partials[:, 1::2].T
    x_pooled = _tc_pool(p_even, p_odd, x4)

    idx = jnp.arange(num_keep, dtype=jnp.int64)
    left = idx[:-1]
    right = idx[1:]
    new_edge_index = jnp.concatenate(
        [jnp.stack([left, right], axis=0), jnp.stack([right, left], axis=0)], axis=1
    )
    return (x_pooled, new_edge_index)
